# baseline (speedup 1.0000x reference)
import functools

import jax
import jax.numpy as jnp
from jax import lax
from jax.experimental import pallas as pl
from jax.experimental.pallas import tpu as pltpu
from jax.experimental.pallas import tpu_sc as plsc

D_MODEL = 64
SCALE = float(D_MODEL) ** 0.5
NUM_WORKERS = 32
CHUNK = 128


def _emb_body(idx_hbm, table_hbm, out_hbm, idx_v, rows_v, sem):
    b_total = idx_hbm.shape[0]
    bpw = b_total // NUM_WORKERS
    nchunks = bpw // CHUNK

    wid = lax.axis_index("s") * 2 + lax.axis_index("c")
    base = wid * bpw

    pltpu.sync_copy(idx_hbm.at[pl.ds(base, bpw)], idx_v)

    def chunk_body(g, carry):
        pltpu.async_copy(
            table_hbm.at[idx_v.at[pl.ds(g * CHUNK, CHUNK)]], rows_v, sem
        ).wait()

        def scale_row(r, c2):
            for j in range(D_MODEL // 16):
                sl = pl.ds(j * 16, 16)
                rows_v[r, sl] = rows_v[r, sl] * SCALE
            return c2

        lax.fori_loop(0, CHUNK, scale_row, 0)

        pltpu.sync_copy(rows_v, out_hbm.at[pl.ds(base + g * CHUNK, CHUNK)])
        return carry

    lax.fori_loop(0, nchunks, chunk_body, 0)


@jax.jit
def kernel(x, embedding):
    b_total = x.shape[0] * x.shape[1]
    idx = x.reshape(b_total).astype(jnp.int32)
    bpw = b_total // NUM_WORKERS

    mesh = plsc.VectorSubcoreMesh(core_axis_name="c", subcore_axis_name="s")
    out = pl.kernel(
        _emb_body,
        out_type=jax.ShapeDtypeStruct((b_total, D_MODEL), jnp.float32),
        mesh=mesh,
        scratch_types=[
            pltpu.VMEM((bpw,), jnp.int32),
            pltpu.VMEM((CHUNK, D_MODEL), jnp.float32),
            pltpu.SemaphoreType.DMA,
        ],
        compiler_params=pltpu.CompilerParams(use_tc_tiling_on_sc=False),
    )(idx, embedding)
    return out.reshape(x.shape[0], x.shape[1], D_MODEL)


# speedup vs baseline: 1.2034x; 1.2034x over previous
import jax
import jax.numpy as jnp
from jax import lax
from jax.experimental import pallas as pl
from jax.experimental.pallas import tpu as pltpu
from jax.experimental.pallas import tpu_sc as plsc

D_MODEL = 64
SCALE = float(D_MODEL) ** 0.5
NUM_WORKERS = 32
CHUNK = 128
NBUF = 4


def _emb_body(idx_hbm, table_hbm, out_hbm, idx_v, rows, sems_g, sems_o):
    b_total = idx_hbm.shape[0]
    bpw = b_total // NUM_WORKERS
    nchunks = bpw // CHUNK
    nblocks = nchunks // NBUF

    wid = lax.axis_index("s") * 2 + lax.axis_index("c")
    base = wid * bpw

    pltpu.sync_copy(idx_hbm.at[pl.ds(base, bpw)], idx_v)

    def gather_desc(c, j):
        return pltpu.make_async_copy(
            table_hbm.at[idx_v.at[pl.ds(c * CHUNK, CHUNK)]], rows[j],
            sems_g[j])

    def out_desc(c, j):
        return pltpu.make_async_copy(
            rows[j], out_hbm.at[pl.ds(base + c * CHUNK, CHUNK)], sems_o[j])

    def scale(j):
        def scale_row(r, c2):
            for k in range(D_MODEL // 16):
                sl = pl.ds(k * 16, 16)
                rows[j][r, sl] = rows[j][r, sl] * SCALE
            return c2

        lax.fori_loop(0, CHUNK, scale_row, 0, unroll=4)

    def step(c, j, do_gather=True, do_wait_out=True):
        jg = (j + 2) % NBUF
        if do_gather:
            if do_wait_out:
                out_desc(c + 2 - NBUF, jg).wait()
            gather_desc(c + 2, jg).start()
        gather_desc(c, j).wait()
        scale(j)
        out_desc(c, j).start()

    gather_desc(0, 0).start()
    gather_desc(1, 1).start()

    step(0, 0, do_wait_out=False)
    step(1, 1, do_wait_out=False)
    step(2, 2)
    step(3, 3)

    def block_body(blk, carry):
        c0 = blk * NBUF
        for j in range(NBUF):
            step(c0 + j, j)
        return carry

    lax.fori_loop(1, nblocks - 1, block_body, 0)

    c0 = (nblocks - 1) * NBUF
    step(c0 + 0, 0)
    step(c0 + 1, 1)
    step(c0 + 2, 2, do_gather=False)
    step(c0 + 3, 3, do_gather=False)

    for j in range(NBUF):
        out_desc(c0 + j, j).wait()


@jax.jit
def kernel(x, embedding):
    b_total = x.shape[0] * x.shape[1]
    idx = x.reshape(b_total).astype(jnp.int32)
    bpw = b_total // NUM_WORKERS

    mesh = plsc.VectorSubcoreMesh(core_axis_name="c", subcore_axis_name="s")
    out = pl.kernel(
        _emb_body,
        out_type=jax.ShapeDtypeStruct((b_total, D_MODEL), jnp.float32),
        mesh=mesh,
        scratch_types=[
            pltpu.VMEM((bpw,), jnp.int32),
            [pltpu.VMEM((CHUNK, D_MODEL), jnp.float32) for _ in range(NBUF)],
            [pltpu.SemaphoreType.DMA for _ in range(NBUF)],
            [pltpu.SemaphoreType.DMA for _ in range(NBUF)],
        ],
        compiler_params=pltpu.CompilerParams(use_tc_tiling_on_sc=False),
    )(idx, embedding)
    return out.reshape(x.shape[0], x.shape[1], D_MODEL)
